# Initial kernel scaffold; baseline (speedup 1.0000x reference)
import jax
import jax.numpy as jnp
from jax.experimental import pallas as pl


def kernel(nodes, edge_attr, globals_attr, edge_index, W_e, b_e, W_n, b_n, W_g, b_g):
    raise NotImplementedError("write your pallas kernel here")


# speedup vs baseline: 3.6278x; 3.6278x over previous
import functools

import jax
import jax.numpy as jnp
from jax import lax
from jax.experimental import pallas as pl
from jax.experimental.pallas import tpu as pltpu
from jax.experimental.pallas import tpu_sc as plsc

NUM_CORES = 2
NUM_SUBCORES = 16
NUM_WORKERS = NUM_CORES * NUM_SUBCORES


def _node_proj_body(nodes_ref, ws_ref, wd_ref, be_ref, psrc_ref, pdst_ref):
    n = nodes_ref[...]
    psrc_ref[...] = jnp.dot(n, ws_ref[...], preferred_element_type=jnp.float32)
    pdst_ref[...] = (
        jnp.dot(n, wd_ref[...], preferred_element_type=jnp.float32) + be_ref[...]
    )


def _node_proj(nodes, w_src, w_dst, b_e, block_rows):
    n_nodes, d = nodes.shape
    h = w_src.shape[1]
    grid = (n_nodes // block_rows,)
    return pl.pallas_call(
        _node_proj_body,
        grid=grid,
        in_specs=[
            pl.BlockSpec((block_rows, d), lambda i: (i, 0)),
            pl.BlockSpec((d, h), lambda i: (0, 0)),
            pl.BlockSpec((d, h), lambda i: (0, 0)),
            pl.BlockSpec((1, h), lambda i: (0, 0)),
        ],
        out_specs=[
            pl.BlockSpec((block_rows, h), lambda i: (i, 0)),
            pl.BlockSpec((block_rows, h), lambda i: (i, 0)),
        ],
        out_shape=[
            jax.ShapeDtypeStruct((n_nodes, h), jnp.float32),
            jax.ShapeDtypeStruct((n_nodes, h), jnp.float32),
        ],
    )(nodes, w_src, w_dst, b_e.reshape(1, h))


def _eproj_body(ea_ref, wa_ref, out_ref):
    out_ref[...] = jnp.dot(
        ea_ref[...], wa_ref[...], preferred_element_type=jnp.float32
    )


def _eproj(edge_attr, w_attr, block_rows):
    e, de = edge_attr.shape
    h = w_attr.shape[1]
    grid = (e // block_rows,)
    return pl.pallas_call(
        _eproj_body,
        grid=grid,
        in_specs=[
            pl.BlockSpec((block_rows, de), lambda i: (i, 0)),
            pl.BlockSpec((de, h), lambda i: (0, 0)),
        ],
        out_specs=pl.BlockSpec((block_rows, h), lambda i: (i, 0)),
        out_shape=jax.ShapeDtypeStruct((e, h), jnp.float32),
    )(edge_attr, w_attr)


def _sc_edge_body(
    n_pad,
    h,
    chunk,
    n_chunks,
    zrows,
    psrc_hbm,
    pdst_hbm,
    eproj_hbm,
    src_hbm,
    dst_hbm,
    out_hbm,
    idx_s,
    idx_d,
    sbuf,
    dbuf,
    abuf,
    zbuf,
    agg_sh,
    sem_s,
    sem_d,
    sem_a,
):
    c = lax.axis_index("c")
    s = lax.axis_index("s")
    wid = s * NUM_CORES + c
    rows_per_tile = n_pad // NUM_SUBCORES
    edges_per_worker = chunk * n_chunks

    zero = jnp.zeros((16,), jnp.float32)

    def _zrow(i, carry):
        for j in range(h // 16):
            zbuf[i, pl.ds(j * 16, 16)] = zero
        return carry

    lax.fori_loop(0, zrows, _zrow, 0)
    for r in range(rows_per_tile // zrows):
        pltpu.sync_copy(
            zbuf, agg_sh.at[pl.ds(s * rows_per_tile + r * zrows, zrows)]
        )
    plsc.subcore_barrier()

    ebase = wid * edges_per_worker

    def _chunk(ci, carry):
        base = ebase + ci * chunk
        pltpu.sync_copy(src_hbm.at[pl.ds(base, chunk)], idx_s)
        pltpu.sync_copy(dst_hbm.at[pl.ds(base, chunk)], idx_d)
        cp_s = pltpu.async_copy(psrc_hbm.at[idx_s], sbuf, sem_s)
        cp_d = pltpu.async_copy(pdst_hbm.at[idx_d], dbuf, sem_d)
        cp_a = pltpu.async_copy(eproj_hbm.at[pl.ds(base, chunk)], abuf, sem_a)
        cp_s.wait()
        cp_d.wait()
        cp_a.wait()

        def _row(e, inner):
            for j in range(h // 16):
                sl = pl.ds(j * 16, 16)
                abuf[e, sl] = jnp.maximum(
                    abuf[e, sl] + sbuf[e, sl] + dbuf[e, sl], 0.0
                )
            return inner

        lax.fori_loop(0, chunk, _row, 0)
        pltpu.sync_copy(abuf, agg_sh.at[idx_d], add=True)
        return carry

    lax.fori_loop(0, n_chunks, _chunk, 0)
    plsc.subcore_barrier()

    out_base = c * n_pad + s * rows_per_tile
    pltpu.sync_copy(
        agg_sh.at[pl.ds(s * rows_per_tile, rows_per_tile)],
        out_hbm.at[pl.ds(out_base, rows_per_tile)],
    )


def _sc_segment_sum(psrc, pdstb, eproj, src, dst, n_pad, chunk, n_chunks,
                    zrows):
    _, h = psrc.shape
    mesh = plsc.VectorSubcoreMesh(
        core_axis_name="c",
        subcore_axis_name="s",
        num_cores=NUM_CORES,
        num_subcores=NUM_SUBCORES,
    )
    body = functools.partial(_sc_edge_body, n_pad, h, chunk, n_chunks, zrows)
    return pl.kernel(
        body,
        out_type=jax.ShapeDtypeStruct((NUM_CORES * n_pad, h), jnp.float32),
        mesh=mesh,
        scratch_types=[
            pltpu.VMEM((chunk,), jnp.int32),
            pltpu.VMEM((chunk,), jnp.int32),
            pltpu.VMEM((chunk, h), jnp.float32),
            pltpu.VMEM((chunk, h), jnp.float32),
            pltpu.VMEM((chunk, h), jnp.float32),
            pltpu.VMEM((zrows, h), jnp.float32),
            pltpu.VMEM_SHARED((n_pad, h), jnp.float32),
            pltpu.SemaphoreType.DMA,
            pltpu.SemaphoreType.DMA,
            pltpu.SemaphoreType.DMA,
        ],
    )(psrc, pdstb, eproj, src, dst)


def _node_global_body(
    n_nodes,
    n_blocks,
    nodes_ref,
    a0_ref,
    a1_ref,
    wn1_ref,
    wn2_ref,
    bn_ref,
    g_ref,
    wg1_ref,
    wg2_ref,
    bg_ref,
    nn_ref,
    gout_ref,
    acc_ref,
):
    i = pl.program_id(0)
    agg = a0_ref[...] + a1_ref[...]
    hid = jnp.dot(nodes_ref[...], wn1_ref[...], preferred_element_type=jnp.float32)
    hid += jnp.dot(agg, wn2_ref[...], preferred_element_type=jnp.float32)
    hid = jnp.maximum(hid + bn_ref[...], 0.0)
    nn_ref[...] = hid
    part = jnp.sum(hid, axis=0, keepdims=True)

    @pl.when(i == 0)
    def _():
        acc_ref[...] = part

    @pl.when(i > 0)
    def _():
        acc_ref[...] = acc_ref[...] + part

    @pl.when(i == n_blocks - 1)
    def _():
        mean8 = jnp.broadcast_to(acc_ref[...] / n_nodes, (8, acc_ref.shape[1]))
        g8 = jnp.broadcast_to(g_ref[...], (8, g_ref.shape[1]))
        out = jnp.dot(mean8, wg1_ref[...], preferred_element_type=jnp.float32)
        out += jnp.dot(g8, wg2_ref[...], preferred_element_type=jnp.float32)
        gout_ref[...] = jnp.maximum(out + bg_ref[...], 0.0)


def _node_global(nodes, agg0, agg1, w_n1, w_n2, b_n, globals_attr, w_g1, w_g2,
                 b_g, block_rows):
    n_nodes, d = nodes.shape
    h = w_n1.shape[1]
    dg = w_g1.shape[1]
    n_blocks = n_nodes // block_rows
    body = functools.partial(_node_global_body, n_nodes, n_blocks)
    return pl.pallas_call(
        body,
        grid=(n_blocks,),
        in_specs=[
            pl.BlockSpec((block_rows, d), lambda i: (i, 0)),
            pl.BlockSpec((block_rows, h), lambda i: (i, 0)),
            pl.BlockSpec((block_rows, h), lambda i: (i, 0)),
            pl.BlockSpec((d, h), lambda i: (0, 0)),
            pl.BlockSpec((h, h), lambda i: (0, 0)),
            pl.BlockSpec((1, h), lambda i: (0, 0)),
            pl.BlockSpec((1, globals_attr.shape[1]), lambda i: (0, 0)),
            pl.BlockSpec((d, dg), lambda i: (0, 0)),
            pl.BlockSpec((globals_attr.shape[1], dg), lambda i: (0, 0)),
            pl.BlockSpec((1, dg), lambda i: (0, 0)),
        ],
        out_specs=[
            pl.BlockSpec((block_rows, h), lambda i: (i, 0)),
            pl.BlockSpec((8, dg), lambda i: (0, 0)),
        ],
        out_shape=[
            jax.ShapeDtypeStruct((n_nodes, h), jnp.float32),
            jax.ShapeDtypeStruct((8, dg), jnp.float32),
        ],
        scratch_shapes=[pltpu.VMEM((1, h), jnp.float32)],
    )(nodes, agg0, agg1, w_n1, w_n2, b_n.reshape(1, h), globals_attr, w_g1,
      w_g2, b_g.reshape(1, dg))


def kernel(nodes, edge_attr, globals_attr, edge_index, W_e, b_e, W_n, b_n,
           W_g, b_g):
    n_nodes, d = nodes.shape
    e_edges, de = edge_attr.shape
    h = W_e.shape[1]
    dg = W_g.shape[1]

    w_src = W_e[:d]
    w_dst = W_e[d : 2 * d]
    w_attr = W_e[2 * d :]
    src = edge_index[0]
    dst = edge_index[1]

    psrc, pdstb = _node_proj(nodes, w_src, w_dst, b_e, block_rows=2000)
    eproj = _eproj(edge_attr, w_attr, block_rows=4000)

    chunk = 80
    n_chunks = e_edges // (NUM_WORKERS * chunk)
    n_pad = ((n_nodes + 8 * NUM_SUBCORES - 1) // (8 * NUM_SUBCORES)) * (
        8 * NUM_SUBCORES
    )
    aggp = _sc_segment_sum(psrc, pdstb, eproj, src, dst, n_pad, chunk,
                           n_chunks, zrows=128)
    agg0 = aggp[:n_nodes]
    agg1 = aggp[n_pad : n_pad + n_nodes]

    new_nodes, g8 = _node_global(
        nodes, agg0, agg1, W_n[:d], W_n[d:], b_n, globals_attr, W_g[:d],
        W_g[d:], b_g, block_rows=2000,
    )
    return new_nodes, edge_attr, g8[:1]
